# baseline (speedup 1.0000x reference)
import functools
import math

import jax
import jax.numpy as jnp
from jax import lax
from jax.experimental import pallas as pl
from jax.experimental.pallas import tpu as pltpu
from jax.experimental.pallas import tpu_sc as plsc

N = 10000
F = 128
E = 320000
POLY_D = 4
NUM_FILTERS = POLY_D + 1
NUM_TERMS = POLY_D + 3
K_STEPS = NUM_TERMS - 1

NUM_CORES = 2
NUM_SUBCORES = 16
NUM_WORKERS = NUM_CORES * NUM_SUBCORES
TN = 640
NP = NUM_SUBCORES * TN
CHUNK = 128
CPT = 80
EPW = CPT * CHUNK
EP = NUM_WORKERS * EPW
DUMMY = NP - 1
DW = 16


def _theta_coeffs(d):
    ev = 1.4
    offset = 2
    thetas = []
    for i in range(offset, d + 1 + offset):
        m = d - i + offset
        B = math.factorial(i) * math.factorial(d + 2 - i) / math.factorial(d + 3)
        coeffs = [0.0] * (d + offset + 1)
        for j in range(m + 1):
            coeffs[i + j] = math.comb(m, j) * ((-1.0 / ev) ** j) / (ev ** i) / (ev * B)
        thetas.append(coeffs)
    return thetas


THETAS = _theta_coeffs(POLY_D)

_MESH = dict(core_axis_name="c", subcore_axis_name="s",
             num_cores=NUM_CORES, num_subcores=NUM_SUBCORES)


def _sc_degree(src_r):
    @functools.partial(
        pl.kernel,
        out_type=jax.ShapeDtypeStruct((NUM_CORES * NP, F), jnp.float32),
        mesh=plsc.VectorSubcoreMesh(**_MESH),
        scratch_types=[
            pltpu.VMEM((CHUNK,), jnp.int32),
            pltpu.VMEM((CHUNK,), jnp.int32),
            pltpu.VMEM((CHUNK, F), jnp.float32),
            pltpu.VMEM_SHARED((NP, F), jnp.float32),
            pltpu.SemaphoreType.DMA,
            pltpu.SemaphoreType.DMA,
        ],
    )
    def deg_kernel(src_hbm, out_hbm, idx0_v, idx1_v, ones_v, acc_sh, is0, is1):
        cid = lax.axis_index("c")
        sid = lax.axis_index("s")
        wid = sid * NUM_CORES + cid
        idxv = (idx0_v, idx1_v)
        isem = (is0, is1)

        one = jnp.full((16,), 1.0, jnp.float32)
        zero = jnp.zeros((16,), jnp.float32)

        def zfill(i, carry):
            for l in range(F // 16):
                ones_v[i, pl.ds(l * 16, 16)] = zero
            return carry

        def fill(i, carry):
            for l in range(F // 16):
                ones_v[i, pl.ds(l * 16, 16)] = one
            return carry

        lax.fori_loop(0, CHUNK, zfill, 0)
        for t in range(TN // CHUNK):
            pltpu.sync_copy(ones_v, acc_sh.at[pl.ds(sid * TN + t * CHUNK, CHUNK)])
        lax.fori_loop(0, CHUNK, fill, 0)
        plsc.subcore_barrier()

        def fire_idx(b, j):
            pltpu.async_copy(src_hbm.at[wid, j], idxv[b], isem[b])

        def wait_idx(b):
            pltpu.make_async_copy(src_hbm.at[0, 0], idxv[b], isem[b]).wait()

        fire_idx(0, 0)

        def outer(io, carry):
            for b in range(2):
                j = io * 2 + b
                nb = 1 - b

                @pl.when(j + 1 < CPT)
                def _():
                    fire_idx(nb, j + 1)

                wait_idx(b)
                pltpu.sync_copy(ones_v, acc_sh.at[idxv[b]], add=True)
            return carry

        lax.fori_loop(0, CPT // 2, outer, 0)

        plsc.subcore_barrier()
        for t in range(TN // CHUNK):
            pltpu.sync_copy(acc_sh.at[pl.ds(sid * TN + t * CHUNK, CHUNK)], ones_v)
            pltpu.sync_copy(
                ones_v, out_hbm.at[pl.ds(cid * NP + sid * TN + t * CHUNK, CHUNK)])

    return deg_kernel(src_r)


def _sc_spmm(g_pad, src_r, dst_r):
    @functools.partial(
        pl.kernel,
        out_type=jax.ShapeDtypeStruct((NUM_CORES * NP, F), jnp.float32),
        mesh=plsc.VectorSubcoreMesh(**_MESH),
        scratch_types=[
            pltpu.VMEM((CPT, CHUNK), jnp.int32),
            pltpu.VMEM((CHUNK,), jnp.int32),
            pltpu.VMEM((CHUNK,), jnp.int32),
            pltpu.VMEM((CHUNK, F), jnp.float32),
            pltpu.VMEM((CHUNK, F), jnp.float32),
            pltpu.VMEM_SHARED((NP, F), jnp.float32),
            pltpu.SemaphoreType.DMA,
            pltpu.SemaphoreType.DMA,
            pltpu.SemaphoreType.DMA,
            pltpu.SemaphoreType.DMA,
        ],
    )
    def spmm_kernel(g_hbm, src_hbm, dst_hbm, out_hbm,
                    src2_v, dsti0_v, dsti1_v, rows0_v, rows1_v,
                    acc_sh, gs0, gs1, ds0, ds1):
        cid = lax.axis_index("c")
        sid = lax.axis_index("s")
        wid = sid * NUM_CORES + cid
        rows = (rows0_v, rows1_v)
        dsti = (dsti0_v, dsti1_v)
        gsem = (gs0, gs1)
        dsem = (ds0, ds1)

        zero = jnp.zeros((16,), jnp.float32)

        def zfill(i, carry):
            for l in range(F // 16):
                rows0_v[i, pl.ds(l * 16, 16)] = zero
            return carry

        lax.fori_loop(0, CHUNK, zfill, 0)
        for t in range(TN // CHUNK):
            pltpu.sync_copy(rows0_v, acc_sh.at[pl.ds(sid * TN + t * CHUNK, CHUNK)])

        pltpu.sync_copy(src_hbm.at[wid], src2_v)
        plsc.subcore_barrier()

        def fire_dst(b, j):
            pltpu.async_copy(dst_hbm.at[wid, j], dsti[b], dsem[b])

        def wait_dst(b):
            pltpu.make_async_copy(dst_hbm.at[0, 0], dsti[b], dsem[b]).wait()

        def fire_gather(b, j):
            pltpu.async_copy(g_hbm.at[src2_v.at[j]], rows[b], gsem[b])

        def wait_gather(b):
            pltpu.make_async_copy(g_hbm.at[pl.ds(0, CHUNK)], rows[b],
                                  gsem[b]).wait()

        fire_gather(0, 0)
        fire_dst(0, 0)

        def outer(io, carry):
            for b in range(2):
                j = io * 2 + b
                nb = 1 - b

                @pl.when(j + 1 < CPT)
                def _():
                    fire_gather(nb, j + 1)
                    fire_dst(nb, j + 1)

                wait_gather(b)
                wait_dst(b)
                pltpu.sync_copy(rows[b], acc_sh.at[dsti[b]], add=True)
            return carry

        lax.fori_loop(0, CPT // 2, outer, 0)

        plsc.subcore_barrier()
        for t in range(TN // CHUNK):
            pltpu.sync_copy(acc_sh.at[pl.ds(sid * TN + t * CHUNK, CHUNK)], rows0_v)
            pltpu.sync_copy(
                rows0_v, out_hbm.at[pl.ds(cid * NP + sid * TN + t * CHUNK, CHUNK)])

    return spmm_kernel(g_pad, src_r, dst_r)


def _tc_matmul(x_p, W, b2):
    def body(x_ref, w_ref, b_ref, h_ref):
        h = jnp.dot(x_ref[...], w_ref[...], preferred_element_type=jnp.float32)
        h = h + b_ref[...]
        h_ref[...] = jnp.where(h >= 0.0, h, 0.01 * h)

    return pl.pallas_call(
        body,
        grid=(NP // TN,),
        in_specs=[
            pl.BlockSpec((TN, F), lambda j: (j, 0)),
            pl.BlockSpec((F, F), lambda j: (0, 0)),
            pl.BlockSpec((1, F), lambda j: (0, 0)),
        ],
        out_specs=pl.BlockSpec((TN, F), lambda j: (j, 0)),
        out_shape=jax.ShapeDtypeStruct((NP, F), jnp.float32),
    )(x_p, W, b2)


def _tc_dsg(h, degp):
    def body(h_ref, d0_ref, d1_ref, g0_ref, ds_ref):
        deg = d0_ref[...][:, :DW] + d1_ref[...][:, :DW]
        dsv = lax.rsqrt(jnp.maximum(deg, 1.0))
        ds_ref[...] = dsv
        g0_ref[...] = dsv[:, :1] * h_ref[...]

    return pl.pallas_call(
        body,
        grid=(NP // TN,),
        in_specs=[
            pl.BlockSpec((TN, F), lambda j: (j, 0)),
            pl.BlockSpec((TN, F), lambda j: (j, 0)),
            pl.BlockSpec((TN, F), lambda j: (j + NUM_SUBCORES, 0)),
        ],
        out_specs=[
            pl.BlockSpec((TN, F), lambda j: (j, 0)),
            pl.BlockSpec((TN, DW), lambda j: (j, 0)),
        ],
        out_shape=[
            jax.ShapeDtypeStruct((NP, F), jnp.float32),
            jax.ShapeDtypeStruct((NP, DW), jnp.float32),
        ],
    )(h, degp, degp)


def _tc_update(p, aggp, ds):
    def body(p_ref, a0_ref, a1_ref, ds_ref, pn_ref, gn_ref):
        agg = a0_ref[...] + a1_ref[...]
        dsv = ds_ref[...][:, :1]
        pn = p_ref[...] - dsv * agg
        pn_ref[...] = pn
        gn_ref[...] = dsv * pn

    return pl.pallas_call(
        body,
        grid=(NP // TN,),
        in_specs=[
            pl.BlockSpec((TN, F), lambda j: (j, 0)),
            pl.BlockSpec((TN, F), lambda j: (j, 0)),
            pl.BlockSpec((TN, F), lambda j: (j + NUM_SUBCORES, 0)),
            pl.BlockSpec((TN, DW), lambda j: (j, 0)),
        ],
        out_specs=[
            pl.BlockSpec((TN, F), lambda j: (j, 0)),
            pl.BlockSpec((TN, F), lambda j: (j, 0)),
        ],
        out_shape=[
            jax.ShapeDtypeStruct((NP, F), jnp.float32),
            jax.ShapeDtypeStruct((NP, F), jnp.float32),
        ],
    )(p, aggp, aggp, ds)


def _tc_combine(ps):
    ROWS = 1000

    def body(*refs):
        p_refs = refs[:NUM_TERMS]
        out_ref = refs[NUM_TERMS]
        vals = [r[...] for r in p_refs]
        for i in range(NUM_FILTERS):
            acc = THETAS[i][0] * vals[0]
            for k in range(1, NUM_TERMS):
                acc = acc + THETAS[i][k] * vals[k]
            out_ref[:, i * F:(i + 1) * F] = acc

    return pl.pallas_call(
        body,
        grid=(N // ROWS,),
        in_specs=[pl.BlockSpec((ROWS, F), lambda j: (j, 0))] * NUM_TERMS,
        out_specs=pl.BlockSpec((ROWS, NUM_FILTERS * F), lambda j: (j, 0)),
        out_shape=jax.ShapeDtypeStruct((N, NUM_FILTERS * F), jnp.float32),
    )(*ps)


def kernel(x, edge_index, W, b):
    src = edge_index[0].astype(jnp.int32)
    dst = edge_index[1].astype(jnp.int32)
    pad_idx = jnp.full((EP - E,), DUMMY, jnp.int32)
    src_r = jnp.concatenate([src, pad_idx]).reshape(NUM_WORKERS, CPT, CHUNK)
    dst_r = jnp.concatenate([dst, pad_idx]).reshape(NUM_WORKERS, CPT, CHUNK)
    x_p = jnp.pad(x, ((0, NP - N), (0, 0)))
    b2 = b.reshape(1, F)

    degp = _sc_degree(src_r)
    h = _tc_matmul(x_p, W, b2)
    g, ds = _tc_dsg(h, degp)

    ps = [h]
    for _ in range(K_STEPS):
        aggp = _sc_spmm(g, src_r, dst_r)
        pn, g = _tc_update(ps[-1], aggp, ds)
        ps.append(pn)
    return _tc_combine(ps)


# speedup vs baseline: 1.9576x; 1.9576x over previous
import functools
import math

import jax
import jax.numpy as jnp
from jax import lax
from jax.experimental import pallas as pl
from jax.experimental.pallas import tpu as pltpu
from jax.experimental.pallas import tpu_sc as plsc

N = 10000
F = 128
E = 320000
POLY_D = 4
NUM_FILTERS = POLY_D + 1
NUM_TERMS = POLY_D + 3
K_STEPS = NUM_TERMS - 1

NUM_CORES = 2
NUM_SUBCORES = 16
NUM_WORKERS = NUM_CORES * NUM_SUBCORES
TN = 640
NP = NUM_SUBCORES * TN
CHUNK = 128
CPT = 80
FAST_CID = 0
CPT_F = 144
CPT_S = 16
NCHUNKS = NUM_SUBCORES * (CPT_F + CPT_S)
NCHUNKS_PAD = NCHUNKS + (CPT_F - CPT_S)
EP = NCHUNKS * CHUNK
DUMMY = NP - 1
DW = 16


def _theta_coeffs(d):
    ev = 1.4
    offset = 2
    thetas = []
    for i in range(offset, d + 1 + offset):
        m = d - i + offset
        B = math.factorial(i) * math.factorial(d + 2 - i) / math.factorial(d + 3)
        coeffs = [0.0] * (d + offset + 1)
        for j in range(m + 1):
            coeffs[i + j] = math.comb(m, j) * ((-1.0 / ev) ** j) / (ev ** i) / (ev * B)
        thetas.append(coeffs)
    return thetas


THETAS = _theta_coeffs(POLY_D)

_MESH = dict(core_axis_name="c", subcore_axis_name="s",
             num_cores=NUM_CORES, num_subcores=NUM_SUBCORES)


def _sc_degree(src_r):
    @functools.partial(
        pl.kernel,
        out_type=jax.ShapeDtypeStruct((NUM_CORES * NP, F), jnp.float32),
        mesh=plsc.VectorSubcoreMesh(**_MESH),
        scratch_types=[
            pltpu.VMEM((CHUNK,), jnp.int32),
            pltpu.VMEM((CHUNK,), jnp.int32),
            pltpu.VMEM((CHUNK, F), jnp.float32),
            pltpu.VMEM_SHARED((NP, F), jnp.float32),
            pltpu.SemaphoreType.DMA,
            pltpu.SemaphoreType.DMA,
        ],
    )
    def deg_kernel(src_hbm, out_hbm, idx0_v, idx1_v, ones_v, acc_sh, is0, is1):
        cid = lax.axis_index("c")
        sid = lax.axis_index("s")
        wid = sid * NUM_CORES + cid
        cbase = wid * CPT
        idxv = (idx0_v, idx1_v)
        isem = (is0, is1)

        one = jnp.full((16,), 1.0, jnp.float32)
        zero = jnp.zeros((16,), jnp.float32)

        def zfill(i, carry):
            for l in range(F // 16):
                ones_v[i, pl.ds(l * 16, 16)] = zero
            return carry

        def fill(i, carry):
            for l in range(F // 16):
                ones_v[i, pl.ds(l * 16, 16)] = one
            return carry

        lax.fori_loop(0, CHUNK, zfill, 0)
        for t in range(TN // CHUNK):
            pltpu.sync_copy(ones_v, acc_sh.at[pl.ds(sid * TN + t * CHUNK, CHUNK)])
        lax.fori_loop(0, CHUNK, fill, 0)
        plsc.subcore_barrier()

        def fire_idx(b, j):
            pltpu.async_copy(src_hbm.at[cbase + j], idxv[b], isem[b])

        def wait_idx(b):
            pltpu.make_async_copy(src_hbm.at[0], idxv[b], isem[b]).wait()

        fire_idx(0, 0)

        def outer(io, carry):
            for b in range(2):
                j = io * 2 + b
                nb = 1 - b

                @pl.when(j + 1 < CPT)
                def _():
                    fire_idx(nb, j + 1)

                wait_idx(b)
                pltpu.sync_copy(ones_v, acc_sh.at[idxv[b]], add=True)
            return carry

        lax.fori_loop(0, CPT // 2, outer, 0)

        plsc.subcore_barrier()
        for t in range(TN // CHUNK):
            pltpu.sync_copy(acc_sh.at[pl.ds(sid * TN + t * CHUNK, CHUNK)], ones_v)
            pltpu.sync_copy(
                ones_v, out_hbm.at[pl.ds(cid * NP + sid * TN + t * CHUNK, CHUNK)])

    return deg_kernel(src_r)


def _sc_spmm(g_pad, src_r, dst_r):
    @functools.partial(
        pl.kernel,
        out_type=jax.ShapeDtypeStruct((NUM_CORES * NP, F), jnp.float32),
        mesh=plsc.VectorSubcoreMesh(**_MESH),
        scratch_types=[
            pltpu.VMEM((CHUNK,), jnp.int32),
            pltpu.VMEM((CHUNK,), jnp.int32),
            pltpu.VMEM((CHUNK,), jnp.int32),
            pltpu.VMEM((CHUNK,), jnp.int32),
            pltpu.VMEM((CHUNK, F), jnp.float32),
            pltpu.VMEM((CHUNK, F), jnp.float32),
            pltpu.VMEM_SHARED((NP, F), jnp.float32),
            pltpu.SemaphoreType.DMA,
            pltpu.SemaphoreType.DMA,
            pltpu.SemaphoreType.DMA,
            pltpu.SemaphoreType.DMA,
            pltpu.SemaphoreType.DMA,
            pltpu.SemaphoreType.DMA,
        ],
    )
    def spmm_kernel(g_hbm, src_hbm, dst_hbm, out_hbm,
                    srci0_v, srci1_v, dsti0_v, dsti1_v, rows0_v, rows1_v,
                    acc_sh, gs0, gs1, ss0, ss1, ds0, ds1):
        cid = lax.axis_index("c")
        sid = lax.axis_index("s")
        is_fast = cid == FAST_CID
        cbase = jnp.where(is_fast, sid * CPT_F,
                          NUM_SUBCORES * CPT_F + sid * CPT_S)
        my_cpt = jnp.where(is_fast, CPT_F, CPT_S)
        rows = (rows0_v, rows1_v)
        srci = (srci0_v, srci1_v)
        dsti = (dsti0_v, dsti1_v)
        gsem = (gs0, gs1)
        ssem = (ss0, ss1)
        dsem = (ds0, ds1)

        zero = jnp.zeros((16,), jnp.float32)

        def zfill(i, carry):
            for l in range(F // 16):
                rows0_v[i, pl.ds(l * 16, 16)] = zero
            return carry

        lax.fori_loop(0, CHUNK, zfill, 0)
        for t in range(TN // CHUNK):
            pltpu.sync_copy(rows0_v, acc_sh.at[pl.ds(sid * TN + t * CHUNK, CHUNK)])

        plsc.subcore_barrier()

        def fire_src(b, j):
            pltpu.async_copy(src_hbm.at[cbase + j], srci[b], ssem[b])

        def wait_src(b):
            pltpu.make_async_copy(src_hbm.at[0], srci[b], ssem[b]).wait()

        def fire_dst(b, j):
            pltpu.async_copy(dst_hbm.at[cbase + j], dsti[b], dsem[b])

        def wait_dst(b):
            pltpu.make_async_copy(dst_hbm.at[0], dsti[b], dsem[b]).wait()

        def fire_gather(b, j):
            pltpu.async_copy(g_hbm.at[srci[b]], rows[b], gsem[b])

        def wait_gather(b):
            pltpu.make_async_copy(g_hbm.at[pl.ds(0, CHUNK)], rows[b],
                                  gsem[b]).wait()

        fire_src(0, 0)
        fire_src(1, 1)
        fire_dst(0, 0)
        fire_dst(1, 1)
        wait_src(0)
        fire_gather(0, 0)

        def outer(io, carry):
            for b in range(2):
                j = io * 2 + b
                nb = 1 - b

                wait_gather(b)

                @pl.when(j + 1 < my_cpt)
                def _():
                    wait_src(nb)
                    fire_gather(nb, j + 1)

                @pl.when(j + 2 < my_cpt)
                def _():
                    fire_src(b, j + 2)

                wait_dst(b)
                pltpu.sync_copy(rows[b], acc_sh.at[dsti[b]], add=True)

                @pl.when(j + 2 < my_cpt)
                def _():
                    fire_dst(b, j + 2)
            return carry

        lax.fori_loop(0, my_cpt // 2, outer, 0)

        plsc.subcore_barrier()
        for t in range(TN // CHUNK):
            pltpu.sync_copy(acc_sh.at[pl.ds(sid * TN + t * CHUNK, CHUNK)], rows0_v)
            pltpu.sync_copy(
                rows0_v, out_hbm.at[pl.ds(cid * NP + sid * TN + t * CHUNK, CHUNK)])

    return spmm_kernel(g_pad, src_r, dst_r)


def _tc_matmul(x_p, W, b2):
    def body(x_ref, w_ref, b_ref, h_ref):
        h = jnp.dot(x_ref[...], w_ref[...], preferred_element_type=jnp.float32)
        h = h + b_ref[...]
        h_ref[...] = jnp.where(h >= 0.0, h, 0.01 * h)

    return pl.pallas_call(
        body,
        grid=(NP // TN,),
        in_specs=[
            pl.BlockSpec((TN, F), lambda j: (j, 0)),
            pl.BlockSpec((F, F), lambda j: (0, 0)),
            pl.BlockSpec((1, F), lambda j: (0, 0)),
        ],
        out_specs=pl.BlockSpec((TN, F), lambda j: (j, 0)),
        out_shape=jax.ShapeDtypeStruct((NP, F), jnp.float32),
    )(x_p, W, b2)


def _tc_dsg(h, degp):
    def body(h_ref, d0_ref, d1_ref, g0_ref, ds_ref):
        deg = d0_ref[...][:, :DW] + d1_ref[...][:, :DW]
        dsv = lax.rsqrt(jnp.maximum(deg, 1.0))
        ds_ref[...] = dsv
        g0_ref[...] = dsv[:, :1] * h_ref[...]

    return pl.pallas_call(
        body,
        grid=(NP // TN,),
        in_specs=[
            pl.BlockSpec((TN, F), lambda j: (j, 0)),
            pl.BlockSpec((TN, F), lambda j: (j, 0)),
            pl.BlockSpec((TN, F), lambda j: (j + NUM_SUBCORES, 0)),
        ],
        out_specs=[
            pl.BlockSpec((TN, F), lambda j: (j, 0)),
            pl.BlockSpec((TN, DW), lambda j: (j, 0)),
        ],
        out_shape=[
            jax.ShapeDtypeStruct((NP, F), jnp.float32),
            jax.ShapeDtypeStruct((NP, DW), jnp.float32),
        ],
    )(h, degp, degp)


def _tc_update(p, aggp, ds):
    def body(p_ref, a0_ref, a1_ref, ds_ref, pn_ref, gn_ref):
        agg = a0_ref[...] + a1_ref[...]
        dsv = ds_ref[...][:, :1]
        pn = p_ref[...] - dsv * agg
        pn_ref[...] = pn
        gn_ref[...] = dsv * pn

    return pl.pallas_call(
        body,
        grid=(NP // TN,),
        in_specs=[
            pl.BlockSpec((TN, F), lambda j: (j, 0)),
            pl.BlockSpec((TN, F), lambda j: (j, 0)),
            pl.BlockSpec((TN, F), lambda j: (j + NUM_SUBCORES, 0)),
            pl.BlockSpec((TN, DW), lambda j: (j, 0)),
        ],
        out_specs=[
            pl.BlockSpec((TN, F), lambda j: (j, 0)),
            pl.BlockSpec((TN, F), lambda j: (j, 0)),
        ],
        out_shape=[
            jax.ShapeDtypeStruct((NP, F), jnp.float32),
            jax.ShapeDtypeStruct((NP, F), jnp.float32),
        ],
    )(p, aggp, aggp, ds)


def _tc_combine(ps):
    ROWS = 1000

    def body(*refs):
        p_refs = refs[:NUM_TERMS]
        out_ref = refs[NUM_TERMS]
        vals = [r[...] for r in p_refs]
        for i in range(NUM_FILTERS):
            acc = THETAS[i][0] * vals[0]
            for k in range(1, NUM_TERMS):
                acc = acc + THETAS[i][k] * vals[k]
            out_ref[:, i * F:(i + 1) * F] = acc

    return pl.pallas_call(
        body,
        grid=(N // ROWS,),
        in_specs=[pl.BlockSpec((ROWS, F), lambda j: (j, 0))] * NUM_TERMS,
        out_specs=pl.BlockSpec((ROWS, NUM_FILTERS * F), lambda j: (j, 0)),
        out_shape=jax.ShapeDtypeStruct((N, NUM_FILTERS * F), jnp.float32),
    )(*ps)


def kernel(x, edge_index, W, b):
    src = edge_index[0].astype(jnp.int32)
    dst = edge_index[1].astype(jnp.int32)
    pad_idx = N + jnp.arange(NCHUNKS_PAD * CHUNK - E, dtype=jnp.int32) % (NP - N)
    src_r = jnp.concatenate([src, pad_idx]).reshape(NCHUNKS_PAD, CHUNK)
    dst_r = jnp.concatenate([dst, pad_idx]).reshape(NCHUNKS_PAD, CHUNK)
    x_p = jnp.pad(x, ((0, NP - N), (0, 0)))
    b2 = b.reshape(1, F)

    degp = _sc_degree(src_r)
    h = _tc_matmul(x_p, W, b2)
    g, ds = _tc_dsg(h, degp)

    ps = [h]
    for _ in range(K_STEPS):
        aggp = _sc_spmm(g, src_r, dst_r)
        pn, g = _tc_update(ps[-1], aggp, ds)
        ps.append(pn)
    return _tc_combine(ps)


# speedup vs baseline: 2.9827x; 1.5237x over previous
import functools
import math

import jax
import jax.numpy as jnp
from jax import lax
from jax.experimental import pallas as pl
from jax.experimental.pallas import tpu as pltpu
from jax.experimental.pallas import tpu_sc as plsc

N = 10000
F = 128
E = 320000
POLY_D = 4
NUM_FILTERS = POLY_D + 1
NUM_TERMS = POLY_D + 3
K_STEPS = NUM_TERMS - 1

NUM_CORES = 2
NUM_SUBCORES = 16
NUM_WORKERS = NUM_CORES * NUM_SUBCORES
TN = 640
NP = NUM_SUBCORES * TN
CHUNK = 128
CPT = 80
FAST_CID = 0
CPT_F = 80
CPT_S = 80
NCHUNKS = NUM_SUBCORES * (CPT_F + CPT_S)
NCHUNKS_PAD = NCHUNKS + (CPT_F - CPT_S)
EP = NCHUNKS * CHUNK
DUMMY = NP - 1
DW = 16


def _theta_coeffs(d):
    ev = 1.4
    offset = 2
    thetas = []
    for i in range(offset, d + 1 + offset):
        m = d - i + offset
        B = math.factorial(i) * math.factorial(d + 2 - i) / math.factorial(d + 3)
        coeffs = [0.0] * (d + offset + 1)
        for j in range(m + 1):
            coeffs[i + j] = math.comb(m, j) * ((-1.0 / ev) ** j) / (ev ** i) / (ev * B)
        thetas.append(coeffs)
    return thetas


THETAS = _theta_coeffs(POLY_D)

_MESH = dict(core_axis_name="c", subcore_axis_name="s",
             num_cores=NUM_CORES, num_subcores=NUM_SUBCORES)


def _sc_degree(src_r):
    @functools.partial(
        pl.kernel,
        out_type=jax.ShapeDtypeStruct((NUM_CORES * NP, F), jnp.float32),
        mesh=plsc.VectorSubcoreMesh(**_MESH),
        scratch_types=[
            pltpu.VMEM((CHUNK,), jnp.int32),
            pltpu.VMEM((CHUNK,), jnp.int32),
            pltpu.VMEM((CHUNK, F), jnp.float32),
            pltpu.VMEM_SHARED((NP, F), jnp.float32),
            pltpu.SemaphoreType.DMA,
            pltpu.SemaphoreType.DMA,
        ],
    )
    def deg_kernel(src_hbm, out_hbm, idx0_v, idx1_v, ones_v, acc_sh, is0, is1):
        cid = lax.axis_index("c")
        sid = lax.axis_index("s")
        wid = sid * NUM_CORES + cid
        cbase = wid * CPT
        idxv = (idx0_v, idx1_v)
        isem = (is0, is1)

        one = jnp.full((16,), 1.0, jnp.float32)
        zero = jnp.zeros((16,), jnp.float32)

        def zfill(i, carry):
            for l in range(F // 16):
                ones_v[i, pl.ds(l * 16, 16)] = zero
            return carry

        def fill(i, carry):
            for l in range(F // 16):
                ones_v[i, pl.ds(l * 16, 16)] = one
            return carry

        lax.fori_loop(0, CHUNK, zfill, 0)
        for t in range(TN // CHUNK):
            pltpu.sync_copy(ones_v, acc_sh.at[pl.ds(sid * TN + t * CHUNK, CHUNK)])
        lax.fori_loop(0, CHUNK, fill, 0)
        plsc.subcore_barrier()

        def fire_idx(b, j):
            pltpu.async_copy(src_hbm.at[cbase + j], idxv[b], isem[b])

        def wait_idx(b):
            pltpu.make_async_copy(src_hbm.at[0], idxv[b], isem[b]).wait()

        fire_idx(0, 0)

        def outer(io, carry):
            for b in range(2):
                j = io * 2 + b
                nb = 1 - b

                @pl.when(j + 1 < CPT)
                def _():
                    fire_idx(nb, j + 1)

                wait_idx(b)
                pltpu.sync_copy(ones_v, acc_sh.at[idxv[b]], add=True)
            return carry

        lax.fori_loop(0, CPT // 2, outer, 0)

        plsc.subcore_barrier()
        for t in range(TN // CHUNK):
            pltpu.sync_copy(acc_sh.at[pl.ds(sid * TN + t * CHUNK, CHUNK)], ones_v)
            pltpu.sync_copy(
                ones_v, out_hbm.at[pl.ds(cid * NP + sid * TN + t * CHUNK, CHUNK)])

    return deg_kernel(src_r)


def _sc_spmm(g_pad, src_r, dst_r):
    @functools.partial(
        pl.kernel,
        out_type=jax.ShapeDtypeStruct((NUM_CORES * NP, F), jnp.float32),
        mesh=plsc.VectorSubcoreMesh(**_MESH),
        scratch_types=[
            pltpu.VMEM((CHUNK,), jnp.int32),
            pltpu.VMEM((CHUNK,), jnp.int32),
            pltpu.VMEM((CHUNK,), jnp.int32),
            pltpu.VMEM((CHUNK,), jnp.int32),
            pltpu.VMEM((CHUNK, F), jnp.float32),
            pltpu.VMEM((CHUNK, F), jnp.float32),
            pltpu.VMEM_SHARED((NP, F), jnp.float32),
            pltpu.SemaphoreType.DMA,
            pltpu.SemaphoreType.DMA,
            pltpu.SemaphoreType.DMA,
            pltpu.SemaphoreType.DMA,
            pltpu.SemaphoreType.DMA,
            pltpu.SemaphoreType.DMA,
        ],
    )
    def spmm_kernel(g_hbm, src_hbm, dst_hbm, out_hbm,
                    srci0_v, srci1_v, dsti0_v, dsti1_v, rows0_v, rows1_v,
                    acc_sh, gs0, gs1, ss0, ss1, ds0, ds1):
        cid = lax.axis_index("c")
        sid = lax.axis_index("s")
        is_fast = cid == FAST_CID
        cbase = jnp.where(is_fast, sid * CPT_F,
                          NUM_SUBCORES * CPT_F + sid * CPT_S)
        my_cpt = jnp.where(is_fast, CPT_F, CPT_S)
        rows = (rows0_v, rows1_v)
        srci = (srci0_v, srci1_v)
        dsti = (dsti0_v, dsti1_v)
        gsem = (gs0, gs1)
        ssem = (ss0, ss1)
        dsem = (ds0, ds1)

        zero = jnp.zeros((16,), jnp.float32)

        def zfill(i, carry):
            for l in range(F // 16):
                rows0_v[i, pl.ds(l * 16, 16)] = zero
            return carry

        lax.fori_loop(0, CHUNK, zfill, 0)
        for t in range(TN // CHUNK):
            pltpu.sync_copy(rows0_v, acc_sh.at[pl.ds(sid * TN + t * CHUNK, CHUNK)])

        plsc.subcore_barrier()

        def fire_src(b, j):
            pltpu.async_copy(src_hbm.at[cbase + j], srci[b], ssem[b])

        def wait_src(b):
            pltpu.make_async_copy(src_hbm.at[0], srci[b], ssem[b]).wait()

        def fire_dst(b, j):
            pltpu.async_copy(dst_hbm.at[cbase + j], dsti[b], dsem[b])

        def wait_dst(b):
            pltpu.make_async_copy(dst_hbm.at[0], dsti[b], dsem[b]).wait()

        def fire_gather(b, j):
            pltpu.async_copy(g_hbm.at[srci[b]], rows[b], gsem[b])

        def wait_gather(b):
            pltpu.make_async_copy(g_hbm.at[pl.ds(0, CHUNK)], rows[b],
                                  gsem[b]).wait()

        fire_src(0, 0)
        fire_src(1, 1)
        fire_dst(0, 0)
        fire_dst(1, 1)
        wait_src(0)
        fire_gather(0, 0)

        def outer(io, carry):
            for b in range(2):
                j = io * 2 + b
                nb = 1 - b

                wait_gather(b)

                @pl.when(j + 1 < my_cpt)
                def _():
                    wait_src(nb)
                    fire_gather(nb, j + 1)

                @pl.when(j + 2 < my_cpt)
                def _():
                    fire_src(b, j + 2)

                wait_dst(b)
                pltpu.sync_copy(rows[b], acc_sh.at[dsti[b]], add=True)

                @pl.when(j + 2 < my_cpt)
                def _():
                    fire_dst(b, j + 2)
            return carry

        lax.fori_loop(0, my_cpt // 2, outer, 0)

        plsc.subcore_barrier()
        for t in range(TN // CHUNK):
            pltpu.sync_copy(acc_sh.at[pl.ds(sid * TN + t * CHUNK, CHUNK)], rows0_v)
            pltpu.sync_copy(
                rows0_v, out_hbm.at[pl.ds(cid * NP + sid * TN + t * CHUNK, CHUNK)])

    return spmm_kernel(g_pad, src_r, dst_r)


def _tc_matmul(x_p, W, b2):
    def body(x_ref, w_ref, b_ref, h_ref):
        h = jnp.dot(x_ref[...], w_ref[...], preferred_element_type=jnp.float32)
        h = h + b_ref[...]
        h_ref[...] = jnp.where(h >= 0.0, h, 0.01 * h)

    return pl.pallas_call(
        body,
        grid=(NP // TN,),
        in_specs=[
            pl.BlockSpec((TN, F), lambda j: (j, 0)),
            pl.BlockSpec((F, F), lambda j: (0, 0)),
            pl.BlockSpec((1, F), lambda j: (0, 0)),
        ],
        out_specs=pl.BlockSpec((TN, F), lambda j: (j, 0)),
        out_shape=jax.ShapeDtypeStruct((NP, F), jnp.float32),
    )(x_p, W, b2)


def _tc_dsg(h, degp):
    def body(h_ref, d0_ref, d1_ref, g0_ref, ds_ref):
        deg = d0_ref[...][:, :DW] + d1_ref[...][:, :DW]
        dsv = lax.rsqrt(jnp.maximum(deg, 1.0))
        ds_ref[...] = dsv
        g0_ref[...] = dsv[:, :1] * h_ref[...]

    return pl.pallas_call(
        body,
        grid=(NP // TN,),
        in_specs=[
            pl.BlockSpec((TN, F), lambda j: (j, 0)),
            pl.BlockSpec((TN, F), lambda j: (j, 0)),
            pl.BlockSpec((TN, F), lambda j: (j + NUM_SUBCORES, 0)),
        ],
        out_specs=[
            pl.BlockSpec((TN, F), lambda j: (j, 0)),
            pl.BlockSpec((TN, DW), lambda j: (j, 0)),
        ],
        out_shape=[
            jax.ShapeDtypeStruct((NP, F), jnp.float32),
            jax.ShapeDtypeStruct((NP, DW), jnp.float32),
        ],
    )(h, degp, degp)


def _tc_update(p, aggp, ds):
    def body(p_ref, a0_ref, a1_ref, ds_ref, pn_ref, gn_ref):
        agg = a0_ref[...] + a1_ref[...]
        dsv = ds_ref[...][:, :1]
        pn = p_ref[...] - dsv * agg
        pn_ref[...] = pn
        gn_ref[...] = dsv * pn

    return pl.pallas_call(
        body,
        grid=(NP // TN,),
        in_specs=[
            pl.BlockSpec((TN, F), lambda j: (j, 0)),
            pl.BlockSpec((TN, F), lambda j: (j, 0)),
            pl.BlockSpec((TN, F), lambda j: (j + NUM_SUBCORES, 0)),
            pl.BlockSpec((TN, DW), lambda j: (j, 0)),
        ],
        out_specs=[
            pl.BlockSpec((TN, F), lambda j: (j, 0)),
            pl.BlockSpec((TN, F), lambda j: (j, 0)),
        ],
        out_shape=[
            jax.ShapeDtypeStruct((NP, F), jnp.float32),
            jax.ShapeDtypeStruct((NP, F), jnp.float32),
        ],
    )(p, aggp, aggp, ds)


def _tc_combine(ps):
    ROWS = 1000

    def body(*refs):
        p_refs = refs[:NUM_TERMS]
        out_ref = refs[NUM_TERMS]
        vals = [r[...] for r in p_refs]
        for i in range(NUM_FILTERS):
            acc = THETAS[i][0] * vals[0]
            for k in range(1, NUM_TERMS):
                acc = acc + THETAS[i][k] * vals[k]
            out_ref[:, i * F:(i + 1) * F] = acc

    return pl.pallas_call(
        body,
        grid=(N // ROWS,),
        in_specs=[pl.BlockSpec((ROWS, F), lambda j: (j, 0))] * NUM_TERMS,
        out_specs=pl.BlockSpec((ROWS, NUM_FILTERS * F), lambda j: (j, 0)),
        out_shape=jax.ShapeDtypeStruct((N, NUM_FILTERS * F), jnp.float32),
    )(*ps)


def kernel(x, edge_index, W, b):
    src = edge_index[0].astype(jnp.int32)
    dst = edge_index[1].astype(jnp.int32)
    pad_idx = N + jnp.arange(NCHUNKS_PAD * CHUNK - E, dtype=jnp.int32) % (NP - N)
    src_r = jnp.concatenate([src, pad_idx]).reshape(NCHUNKS_PAD, CHUNK)
    dst_r = jnp.concatenate([dst, pad_idx]).reshape(NCHUNKS_PAD, CHUNK)
    x_p = jnp.pad(x, ((0, NP - N), (0, 0)))
    b2 = b.reshape(1, F)

    degp = _sc_degree(src_r)
    h = _tc_matmul(x_p, W, b2)
    g, ds = _tc_dsg(h, degp)

    ps = [h]
    for _ in range(K_STEPS):
        aggp = _sc_spmm(g, src_r, dst_r)
        pn, g = _tc_update(ps[-1], aggp, ds)
        ps.append(pn)
    return _tc_combine(ps)


# speedup vs baseline: 3.9716x; 1.3316x over previous
import functools
import math

import jax
import jax.numpy as jnp
from jax import lax
from jax.experimental import pallas as pl
from jax.experimental.pallas import tpu as pltpu
from jax.experimental.pallas import tpu_sc as plsc

N = 10000
F = 128
E = 320000
POLY_D = 4
NUM_FILTERS = POLY_D + 1
NUM_TERMS = POLY_D + 3
K_STEPS = NUM_TERMS - 1

NUM_CORES = 2
NUM_SUBCORES = 16
NUM_WORKERS = NUM_CORES * NUM_SUBCORES
TN = 632
NP = NUM_SUBCORES * TN
CHUNK = 128
CPT = 81
NBUF = 3
NCHUNKS = NUM_WORKERS * CPT
EP = NCHUNKS * CHUNK
DW = 16
RB = NP // 4


def _theta_coeffs(d):
    ev = 1.4
    offset = 2
    thetas = []
    for i in range(offset, d + 1 + offset):
        m = d - i + offset
        B = math.factorial(i) * math.factorial(d + 2 - i) / math.factorial(d + 3)
        coeffs = [0.0] * (d + offset + 1)
        for j in range(m + 1):
            coeffs[i + j] = math.comb(m, j) * ((-1.0 / ev) ** j) / (ev ** i) / (ev * B)
        thetas.append(coeffs)
    return thetas


THETAS = _theta_coeffs(POLY_D)

_MESH = dict(core_axis_name="c", subcore_axis_name="s",
             num_cores=NUM_CORES, num_subcores=NUM_SUBCORES)

_NODE_BLOCKS = [(t * CHUNK, CHUNK) for t in range(TN // CHUNK)]
if TN % CHUNK:
    _NODE_BLOCKS.append((TN - TN % CHUNK, TN % CHUNK))


def _sc_degree(src_r):
    @functools.partial(
        pl.kernel,
        out_type=jax.ShapeDtypeStruct((NUM_CORES * NP, F), jnp.float32),
        mesh=plsc.VectorSubcoreMesh(**_MESH),
        scratch_types=[
            pltpu.VMEM((CHUNK,), jnp.int32),
            pltpu.VMEM((CHUNK,), jnp.int32),
            pltpu.VMEM((CHUNK, F), jnp.float32),
            pltpu.VMEM_SHARED((NP, F), jnp.float32),
            pltpu.SemaphoreType.DMA,
            pltpu.SemaphoreType.DMA,
        ],
    )
    def deg_kernel(src_hbm, out_hbm, idx0_v, idx1_v, ones_v, acc_sh, is0, is1):
        cid = lax.axis_index("c")
        sid = lax.axis_index("s")
        wid = sid * NUM_CORES + cid
        cbase = wid * CPT
        idxv = (idx0_v, idx1_v)
        isem = (is0, is1)

        one = jnp.full((16,), 1.0, jnp.float32)
        zero = jnp.zeros((16,), jnp.float32)

        def zfill(i, carry):
            for l in range(F // 16):
                ones_v[i, pl.ds(l * 16, 16)] = zero
            return carry

        def fill(i, carry):
            for l in range(F // 16):
                ones_v[i, pl.ds(l * 16, 16)] = one
            return carry

        lax.fori_loop(0, CHUNK, zfill, 0)
        for off, sz in _NODE_BLOCKS:
            pltpu.sync_copy(ones_v.at[pl.ds(0, sz)],
                            acc_sh.at[pl.ds(sid * TN + off, sz)])
        lax.fori_loop(0, CHUNK, fill, 0)
        plsc.subcore_barrier()

        def fire_idx(b, j):
            pltpu.async_copy(src_hbm.at[cbase + j], idxv[b], isem[b])

        def wait_idx(b):
            pltpu.make_async_copy(src_hbm.at[0], idxv[b], isem[b]).wait()

        fire_idx(0, 0)

        def outer(io, carry):
            for b in range(2):
                j = io * 2 + b
                nb = 1 - b

                @pl.when(j + 1 < CPT)
                def _():
                    fire_idx(nb, j + 1)

                wait_idx(b)
                pltpu.sync_copy(ones_v, acc_sh.at[idxv[b]], add=True)
            return carry

        lax.fori_loop(0, CPT // 2, outer, 0)
        if CPT % 2:
            wait_idx(0)
            pltpu.sync_copy(ones_v, acc_sh.at[idxv[0]], add=True)

        plsc.subcore_barrier()
        for off, sz in _NODE_BLOCKS:
            pltpu.sync_copy(acc_sh.at[pl.ds(sid * TN + off, sz)],
                            ones_v.at[pl.ds(0, sz)])
            pltpu.sync_copy(
                ones_v.at[pl.ds(0, sz)],
                out_hbm.at[pl.ds(cid * NP + sid * TN + off, sz)])

    return deg_kernel(src_r)


def _sc_spmm(g_pad, src_r, dst_r):
    @functools.partial(
        pl.kernel,
        out_type=jax.ShapeDtypeStruct((NUM_CORES * NP, F), jnp.float32),
        mesh=plsc.VectorSubcoreMesh(**_MESH),
        scratch_types=(
            [pltpu.VMEM((CHUNK,), jnp.int32)] * (2 * NBUF)
            + [pltpu.VMEM((CHUNK, F), jnp.float32)] * NBUF
            + [pltpu.VMEM_SHARED((NP, F), jnp.float32)]
            + [pltpu.SemaphoreType.DMA] * (3 * NBUF)
        ),
    )
    def spmm_kernel(g_hbm, src_hbm, dst_hbm, out_hbm, *scratch):
        srci = scratch[0:NBUF]
        dsti = scratch[NBUF:2 * NBUF]
        rows = scratch[2 * NBUF:3 * NBUF]
        acc_sh = scratch[3 * NBUF]
        gsem = scratch[3 * NBUF + 1:3 * NBUF + 1 + NBUF]
        ssem = scratch[3 * NBUF + 1 + NBUF:3 * NBUF + 1 + 2 * NBUF]
        dsem = scratch[3 * NBUF + 1 + 2 * NBUF:3 * NBUF + 1 + 3 * NBUF]
        cid = lax.axis_index("c")
        sid = lax.axis_index("s")
        wid = sid * NUM_CORES + cid
        cbase = wid * CPT

        zero = jnp.zeros((16,), jnp.float32)
        rows0_v = rows[0]

        def zfill(i, carry):
            for l in range(F // 16):
                rows0_v[i, pl.ds(l * 16, 16)] = zero
            return carry

        lax.fori_loop(0, CHUNK, zfill, 0)
        for off, sz in _NODE_BLOCKS:
            pltpu.sync_copy(rows0_v.at[pl.ds(0, sz)],
                            acc_sh.at[pl.ds(sid * TN + off, sz)])

        plsc.subcore_barrier()

        def fire_src(b, j):
            pltpu.async_copy(src_hbm.at[cbase + j], srci[b], ssem[b])

        def wait_src(b):
            pltpu.make_async_copy(src_hbm.at[0], srci[b], ssem[b]).wait()

        def fire_dst(b, j):
            pltpu.async_copy(dst_hbm.at[cbase + j], dsti[b], dsem[b])

        def wait_dst(b):
            pltpu.make_async_copy(dst_hbm.at[0], dsti[b], dsem[b]).wait()

        def fire_gather(b):
            pltpu.async_copy(g_hbm.at[srci[b]], rows[b], gsem[b])

        def wait_gather(b):
            pltpu.make_async_copy(g_hbm.at[pl.ds(0, CHUNK)], rows[b],
                                  gsem[b]).wait()

        for b in range(NBUF):
            fire_src(b, b)
            fire_dst(b, b)
        for b in range(NBUF - 1):
            wait_src(b)
            fire_gather(b)

        def outer(io, carry):
            for b in range(NBUF):
                j = io * NBUF + b
                b2 = (b + NBUF - 1) % NBUF

                wait_gather(b)

                @pl.when(j + NBUF - 1 < CPT)
                def _():
                    wait_src(b2)
                    fire_gather(b2)

                @pl.when(j + NBUF < CPT)
                def _():
                    fire_src(b, j + NBUF)

                wait_dst(b)
                pltpu.sync_copy(rows[b], acc_sh.at[dsti[b]], add=True)

                @pl.when(j + NBUF < CPT)
                def _():
                    fire_dst(b, j + NBUF)
            return carry

        lax.fori_loop(0, CPT // NBUF, outer, 0)

        plsc.subcore_barrier()
        for off, sz in _NODE_BLOCKS:
            pltpu.sync_copy(acc_sh.at[pl.ds(sid * TN + off, sz)],
                            rows0_v.at[pl.ds(0, sz)])
            pltpu.sync_copy(
                rows0_v.at[pl.ds(0, sz)],
                out_hbm.at[pl.ds(cid * NP + sid * TN + off, sz)])

    return spmm_kernel(g_pad, src_r, dst_r)


def _tc_matmul(x_p, W, b2):
    def body(x_ref, w_ref, b_ref, h_ref):
        h = jnp.dot(x_ref[...], w_ref[...], preferred_element_type=jnp.float32)
        h = h + b_ref[...]
        h_ref[...] = jnp.where(h >= 0.0, h, 0.01 * h)

    return pl.pallas_call(
        body,
        grid=(NP // RB,),
        in_specs=[
            pl.BlockSpec((RB, F), lambda j: (j, 0)),
            pl.BlockSpec((F, F), lambda j: (0, 0)),
            pl.BlockSpec((1, F), lambda j: (0, 0)),
        ],
        out_specs=pl.BlockSpec((RB, F), lambda j: (j, 0)),
        out_shape=jax.ShapeDtypeStruct((NP, F), jnp.float32),
    )(x_p, W, b2)


def _tc_dsg(h, degp):
    def body(h_ref, d0_ref, d1_ref, g0_ref, ds_ref):
        deg = d0_ref[...][:, :DW] + d1_ref[...][:, :DW]
        dsv = lax.rsqrt(jnp.maximum(deg, 1.0))
        ds_ref[...] = dsv
        g0_ref[...] = dsv[:, :1] * h_ref[...]

    return pl.pallas_call(
        body,
        grid=(NP // RB,),
        in_specs=[
            pl.BlockSpec((RB, F), lambda j: (j, 0)),
            pl.BlockSpec((RB, F), lambda j: (j, 0)),
            pl.BlockSpec((RB, F), lambda j: (j + NP // RB, 0)),
        ],
        out_specs=[
            pl.BlockSpec((RB, F), lambda j: (j, 0)),
            pl.BlockSpec((RB, DW), lambda j: (j, 0)),
        ],
        out_shape=[
            jax.ShapeDtypeStruct((NP, F), jnp.float32),
            jax.ShapeDtypeStruct((NP, DW), jnp.float32),
        ],
    )(h, degp, degp)


def _tc_update(p, aggp, ds):
    def body(p_ref, a0_ref, a1_ref, ds_ref, pn_ref, gn_ref):
        agg = a0_ref[...] + a1_ref[...]
        dsv = ds_ref[...][:, :1]
        pn = p_ref[...] - dsv * agg
        pn_ref[...] = pn
        gn_ref[...] = dsv * pn

    return pl.pallas_call(
        body,
        grid=(NP // RB,),
        in_specs=[
            pl.BlockSpec((RB, F), lambda j: (j, 0)),
            pl.BlockSpec((RB, F), lambda j: (j, 0)),
            pl.BlockSpec((RB, F), lambda j: (j + NP // RB, 0)),
            pl.BlockSpec((RB, DW), lambda j: (j, 0)),
        ],
        out_specs=[
            pl.BlockSpec((RB, F), lambda j: (j, 0)),
            pl.BlockSpec((RB, F), lambda j: (j, 0)),
        ],
        out_shape=[
            jax.ShapeDtypeStruct((NP, F), jnp.float32),
            jax.ShapeDtypeStruct((NP, F), jnp.float32),
        ],
    )(p, aggp, aggp, ds)


def _tc_combine(ps):
    ROWS = 2000

    def body(*refs):
        p_refs = refs[:NUM_TERMS]
        out_ref = refs[NUM_TERMS]
        vals = [r[...] for r in p_refs]
        for i in range(NUM_FILTERS):
            acc = THETAS[i][0] * vals[0]
            for k in range(1, NUM_TERMS):
                acc = acc + THETAS[i][k] * vals[k]
            out_ref[:, i * F:(i + 1) * F] = acc

    return pl.pallas_call(
        body,
        grid=(N // ROWS,),
        in_specs=[pl.BlockSpec((ROWS, F), lambda j: (j, 0))] * NUM_TERMS,
        out_specs=pl.BlockSpec((ROWS, NUM_FILTERS * F), lambda j: (j, 0)),
        out_shape=jax.ShapeDtypeStruct((N, NUM_FILTERS * F), jnp.float32),
    )(*ps)


def kernel(x, edge_index, W, b):
    src = edge_index[0].astype(jnp.int32)
    dst = edge_index[1].astype(jnp.int32)
    pad_idx = N + jnp.arange(EP - E, dtype=jnp.int32) % (NP - N)
    src_r = jnp.concatenate([src, pad_idx]).reshape(NCHUNKS, CHUNK)
    dst_r = jnp.concatenate([dst, pad_idx]).reshape(NCHUNKS, CHUNK)
    x_p = jnp.pad(x, ((0, NP - N), (0, 0)))
    b2 = b.reshape(1, F)

    degp = _sc_degree(src_r)
    h = _tc_matmul(x_p, W, b2)
    g, ds = _tc_dsg(h, degp)

    ps = [h]
    for _ in range(K_STEPS):
        aggp = _sc_spmm(g, src_r, dst_r)
        pn, g = _tc_update(ps[-1], aggp, ds)
        ps.append(pn)
    return _tc_combine(ps)
